# Initial kernel scaffold; baseline (speedup 1.0000x reference)
import jax
import jax.numpy as jnp
from jax.experimental import pallas as pl


def kernel(position_ids, embeddings_table):
    raise NotImplementedError("write your pallas kernel here")


# speedup vs baseline: 1.8163x; 1.8163x over previous
import functools

import jax
import jax.numpy as jnp
from jax import lax
from jax.experimental import pallas as pl
from jax.experimental.pallas import tpu as pltpu
from jax.experimental.pallas import tpu_sc as plsc

D = 1024
N_TOTAL = 32768
NW = 32
BPW = N_TOTAL // NW
C = 32
NCHUNK = BPW // C
NPAIR = NCHUNK // 2


def _make_gather():
    mesh = plsc.VectorSubcoreMesh(core_axis_name="c", subcore_axis_name="s")

    @functools.partial(
        pl.kernel,
        mesh=mesh,
        out_type=jax.ShapeDtypeStruct((N_TOTAL, D), jnp.float32),
        scratch_types=[
            pltpu.VMEM((C,), jnp.int32),
            pltpu.VMEM((C, D), jnp.float32),
            pltpu.SemaphoreType.DMA,
        ],
    )
    def gather_kernel(idx_hbm, table_hbm, out_hbm, idx_v, rows_v, sem_g):
        wid = lax.axis_index("s") * 2 + lax.axis_index("c")
        base = wid * BPW

        def body(g, carry):
            pltpu.sync_copy(idx_hbm.at[pl.ds(base + g * C, C)], idx_v)
            pltpu.async_copy(table_hbm.at[idx_v], rows_v, sem_g).wait()
            pltpu.sync_copy(rows_v, out_hbm.at[pl.ds(base + g * C, C)])
            return carry

        lax.fori_loop(0, NCHUNK, body, 0)

    return gather_kernel


_gather = _make_gather()


def kernel(position_ids, embeddings_table):
    ids_flat = position_ids.reshape(-1)
    out = _gather(ids_flat, embeddings_table)
    return out.reshape(position_ids.shape + (D,))


# speedup vs baseline: 2.1520x; 1.1849x over previous
import functools

import jax
import jax.numpy as jnp
from jax import lax
from jax.experimental import pallas as pl
from jax.experimental.pallas import tpu as pltpu
from jax.experimental.pallas import tpu_sc as plsc

D = 1024
N_TOTAL = 32768
NW = 32
BPW = N_TOTAL // NW
C = 32
NCHUNK = BPW // C
NPAIR = NCHUNK // 2


def _make_gather():
    mesh = plsc.VectorSubcoreMesh(core_axis_name="c", subcore_axis_name="s")

    @functools.partial(
        pl.kernel,
        mesh=mesh,
        out_type=jax.ShapeDtypeStruct((N_TOTAL, D), jnp.float32),
        scratch_types=[
            pltpu.VMEM((C,), jnp.int32),
            pltpu.VMEM((C,), jnp.int32),
            pltpu.VMEM((C, D), jnp.float32),
            pltpu.VMEM((C, D), jnp.float32),
            pltpu.SemaphoreType.DMA,
            pltpu.SemaphoreType.DMA,
            pltpu.SemaphoreType.DMA,
            pltpu.SemaphoreType.DMA,
        ],
    )
    def gather_kernel(idx_hbm, table_hbm, out_hbm,
                      idx0, idx1, rows0, rows1, sem_g0, sem_g1, sem_o0, sem_o1):
        wid = lax.axis_index("s") * 2 + lax.axis_index("c")
        base = wid * BPW
        idx = (idx0, idx1)
        rows = (rows0, rows1)
        sem_g = (sem_g0, sem_g1)
        sem_o = (sem_o0, sem_o1)

        def start_gather(g, b):
            pltpu.sync_copy(idx_hbm.at[pl.ds(base + g * C, C)], idx[b])
            pltpu.async_copy(table_hbm.at[idx[b]], rows[b], sem_g[b])

        def wait_gather(b):
            pltpu.make_async_copy(
                table_hbm.at[idx[b]], rows[b], sem_g[b]
            ).wait()

        def start_out(g, b):
            pltpu.async_copy(
                rows[b], out_hbm.at[pl.ds(base + g * C, C)], sem_o[b]
            )

        def wait_out(b):
            pltpu.make_async_copy(
                table_hbm.at[pl.ds(0, C)],
                out_hbm.at[pl.ds(base, C)],
                sem_o[b],
            ).wait()

        def step(g, b, first):
            if not first:
                wait_out(b)
            start_gather(g, b)
            wait_gather(b)
            start_out(g, b)

        step(0, 0, True)
        step(1, 1, True)

        def pair_body(p, carry):
            g0 = 2 * p
            step(g0, 0, False)
            step(g0 + 1, 1, False)
            return carry

        lax.fori_loop(1, NPAIR, pair_body, 0)
        wait_out(0)
        wait_out(1)

    return gather_kernel


_gather = _make_gather()


def kernel(position_ids, embeddings_table):
    ids_flat = position_ids.reshape(-1)
    out = _gather(ids_flat, embeddings_table)
    return out.reshape(position_ids.shape + (D,))


# speedup vs baseline: 2.3147x; 1.0756x over previous
import functools

import jax
import jax.numpy as jnp
from jax import lax
from jax.experimental import pallas as pl
from jax.experimental.pallas import tpu as pltpu
from jax.experimental.pallas import tpu_sc as plsc

D = 1024
N_TOTAL = 32768
NW = 32
BPW = N_TOTAL // NW
C = 32
NCHUNK = BPW // C
NPAIR = NCHUNK // 2


def _make_gather():
    mesh = plsc.VectorSubcoreMesh(core_axis_name="c", subcore_axis_name="s")

    @functools.partial(
        pl.kernel,
        mesh=mesh,
        out_type=jax.ShapeDtypeStruct((N_TOTAL, D), jnp.float32),
        scratch_types=[
            pltpu.VMEM((BPW,), jnp.int32),
            pltpu.VMEM((C, D), jnp.float32),
            pltpu.VMEM((C, D), jnp.float32),
            pltpu.SemaphoreType.DMA,
            pltpu.SemaphoreType.DMA,
            pltpu.SemaphoreType.DMA,
            pltpu.SemaphoreType.DMA,
        ],
    )
    def gather_kernel(idx_hbm, table_hbm, out_hbm,
                      idx_v, rows0, rows1, sem_g0, sem_g1, sem_o0, sem_o1):
        wid = lax.axis_index("s") * 2 + lax.axis_index("c")
        base = wid * BPW
        rows = (rows0, rows1)
        sem_g = (sem_g0, sem_g1)
        sem_o = (sem_o0, sem_o1)

        pltpu.sync_copy(idx_hbm.at[pl.ds(base, BPW)], idx_v)

        def start_gather(g, b):
            pltpu.async_copy(
                table_hbm.at[idx_v.at[pl.ds(g * C, C)]], rows[b], sem_g[b]
            )

        def wait_gather(b):
            pltpu.make_async_copy(
                table_hbm.at[idx_v.at[pl.ds(0, C)]], rows[b], sem_g[b]
            ).wait()

        def start_out(g, b):
            pltpu.async_copy(
                rows[b], out_hbm.at[pl.ds(base + g * C, C)], sem_o[b]
            )

        def wait_out(b):
            pltpu.make_async_copy(
                table_hbm.at[pl.ds(0, C)],
                out_hbm.at[pl.ds(base, C)],
                sem_o[b],
            ).wait()

        def step(g, b, first):
            if not first:
                wait_out(b)
            start_gather(g, b)
            wait_gather(b)
            start_out(g, b)

        step(0, 0, True)
        step(1, 1, True)

        def pair_body(p, carry):
            g0 = 2 * p
            step(g0, 0, False)
            step(g0 + 1, 1, False)
            return carry

        lax.fori_loop(1, NPAIR, pair_body, 0)
        wait_out(0)
        wait_out(1)

    return gather_kernel


_gather = _make_gather()


def kernel(position_ids, embeddings_table):
    ids_flat = position_ids.reshape(-1)
    out = _gather(ids_flat, embeddings_table)
    return out.reshape(position_ids.shape + (D,))


# speedup vs baseline: 2.3166x; 1.0008x over previous
import functools

import jax
import jax.numpy as jnp
from jax import lax
from jax.experimental import pallas as pl
from jax.experimental.pallas import tpu as pltpu
from jax.experimental.pallas import tpu_sc as plsc

D = 1024
N_TOTAL = 32768
NW = 32
BPW = N_TOTAL // NW
C0 = 64
C1 = 32
_SIZES = [C0, C1] * 10 + [C0]
_OFFS = [sum(_SIZES[:i]) for i in range(len(_SIZES))]
assert sum(_SIZES) == BPW and all(o % 32 == 0 for o in _OFFS)


def _make_gather():
    mesh = plsc.VectorSubcoreMesh(core_axis_name="c", subcore_axis_name="s")

    @functools.partial(
        pl.kernel,
        mesh=mesh,
        out_type=jax.ShapeDtypeStruct((N_TOTAL, D), jnp.float32),
        scratch_types=[
            pltpu.VMEM((BPW,), jnp.int32),
            pltpu.VMEM((C0, D), jnp.float32),
            pltpu.VMEM((C1, D), jnp.float32),
            pltpu.SemaphoreType.DMA,
            pltpu.SemaphoreType.DMA,
            pltpu.SemaphoreType.DMA,
        ],
    )
    def gather_kernel(idx_hbm, table_hbm, out_hbm,
                      idx_v, rows0, rows1, sem_g, sem_o0, sem_o1):
        wid = lax.axis_index("s") * 2 + lax.axis_index("c")
        base = wid * BPW
        rows = (rows0, rows1)
        sem_o = (sem_o0, sem_o1)

        pltpu.sync_copy(idx_hbm.at[pl.ds(base, BPW)], idx_v)

        def wait_out(b, n):
            pltpu.make_async_copy(
                table_hbm.at[pl.ds(0, n)],
                out_hbm.at[pl.ds(base, n)],
                sem_o[b],
            ).wait()

        def buf(b, n):
            r = rows[b]
            return r if n == r.shape[0] else r.at[pl.ds(0, n)]

        for i, (off, n) in enumerate(zip(_OFFS, _SIZES)):
            b = i % 2
            if i >= 2:
                wait_out(b, _SIZES[i - 2])
            pltpu.async_copy(
                table_hbm.at[idx_v.at[pl.ds(off, n)]], buf(b, n), sem_g
            ).wait()
            pltpu.async_copy(
                buf(b, n), out_hbm.at[pl.ds(base + off, n)], sem_o[b]
            )

        wait_out(0, _SIZES[-1])
        wait_out(1, _SIZES[-2])

    return gather_kernel


_gather = _make_gather()


def kernel(position_ids, embeddings_table):
    ids_flat = position_ids.reshape(-1)
    out = _gather(ids_flat, embeddings_table)
    return out.reshape(position_ids.shape + (D,))
